# baseline (speedup 1.0000x reference)
import functools
import math

import jax
import jax.numpy as jnp
from jax import lax
from jax.experimental import pallas as pl
from jax.experimental.pallas import tpu as pltpu
from jax.experimental.pallas import tpu_sc as plsc

N, DIN, H, DOUT, E, L = 10000, 128, 128, 64, 320000, 8
ALPHA, THETA = 0.1, 0.5

NP = 10240
NC, NS = 2, 16
NW = NC * NS
K = 128
NCH = 80
EP = NW * NCH * K
RPS = NP // NS
RB = 1024


def _sc_mesh():
    return plsc.VectorSubcoreMesh(core_axis_name="c", subcore_axis_name="s")


def _sc_degree(col_w):
    @functools.partial(
        pl.kernel,
        out_type=jax.ShapeDtypeStruct((NC * NP,), jnp.float32),
        mesh=_sc_mesh(),
        scratch_types=[
            pltpu.VMEM((NCH, K), jnp.int32),
            pltpu.VMEM((K,), jnp.float32),
            pltpu.VMEM((RPS,), jnp.float32),
            pltpu.VMEM_SHARED((NP,), jnp.float32),
            pltpu.SemaphoreType.DMA,
        ],
    )
    def degk(col_hbm, out_hbm, col_v, ones_v, z_v, acc, sem):
        c = lax.axis_index("c")
        s = lax.axis_index("s")
        w = c * NS + s
        pltpu.sync_copy(col_hbm.at[w], col_v)
        for b in range(K // 16):
            ones_v[pl.ds(b * 16, 16)] = jnp.ones((16,), jnp.float32)
        for b in range(RPS // 16):
            z_v[pl.ds(b * 16, 16)] = jnp.zeros((16,), jnp.float32)
        pltpu.sync_copy(z_v, acc.at[pl.ds(s * RPS, RPS)])
        plsc.subcore_barrier()

        def _step(j, _):
            pltpu.async_copy(ones_v, acc.at[col_v.at[j]], sem, add=True)
            return ()

        lax.fori_loop(0, NCH, _step, ())
        pltpu.make_async_copy(col_hbm.at[w], col_v, sem).wait()
        plsc.subcore_barrier()
        pltpu.sync_copy(acc.at[pl.ds(s * RPS, RPS)],
                        out_hbm.at[pl.ds(c * NP + s * RPS, RPS)])

    return degk(col_w)


HH = H // 2
NCH2 = EP // (NS * K)
G = 40


def _sc_scatter_partials(hs0, hs1, row_w, col_w):
    @functools.partial(
        pl.kernel,
        out_type=jax.ShapeDtypeStruct((NC, NP, HH), jnp.float32),
        mesh=_sc_mesh(),
        scratch_types=[
            pltpu.VMEM((G, K), jnp.int32),
            pltpu.VMEM((G, K), jnp.int32),
            pltpu.VMEM((2, K, HH), jnp.float32),
            pltpu.VMEM_SHARED((NP, HH), jnp.float32),
            pltpu.VMEM_SHARED((NP, HH), jnp.float32),
            pltpu.SemaphoreType.DMA,
            pltpu.SemaphoreType.DMA,
        ],
        compiler_params=pltpu.CompilerParams(use_tc_tiling_on_sc=False),
    )
    def scat(hs0_hbm, hs1_hbm, row_hbm, col_hbm, out_hbm, row_v, col_v, buf,
             stage, acc, sem0, sem1):
        c = lax.axis_index("c")
        s = lax.axis_index("s")

        @pl.when(c == 0)
        def _():
            pltpu.sync_copy(hs0_hbm.at[pl.ds(s * RPS, RPS)],
                            stage.at[pl.ds(s * RPS, RPS)])

        @pl.when(c != 0)
        def _():
            pltpu.sync_copy(hs1_hbm.at[pl.ds(s * RPS, RPS)],
                            stage.at[pl.ds(s * RPS, RPS)])

        def _zrow(i, _):
            for b in range(HH // 16):
                buf[0, i, pl.ds(b * 16, 16)] = jnp.zeros((16,), jnp.float32)
            return ()

        lax.fori_loop(0, K, _zrow, ())
        for t in range(RPS // K):
            pltpu.sync_copy(buf.at[0], acc.at[pl.ds(s * RPS + t * K, K)])
        plsc.subcore_barrier()

        def _group(g, _):
            pltpu.sync_copy(row_hbm.at[s, pl.ds(g * G, G)], row_v)
            pltpu.sync_copy(col_hbm.at[s, pl.ds(g * G, G)], col_v)
            pltpu.async_copy(stage.at[row_v.at[0]], buf.at[0], sem0)
            pltpu.async_copy(stage.at[row_v.at[1]], buf.at[1], sem1)

            def _step(k2, _):
                j = 2 * k2
                for b, sem in ((0, sem0), (1, sem1)):
                    jj = j + b
                    pltpu.make_async_copy(stage.at[row_v.at[jj]], buf.at[b],
                                          sem).wait()
                    pltpu.sync_copy(buf.at[b], acc.at[col_v.at[jj]], add=True)

                    @pl.when(jj + 2 < G)
                    def _():
                        pltpu.async_copy(stage.at[row_v.at[jj + 2]],
                                         buf.at[b], sem)
                return ()

            lax.fori_loop(0, G // 2, _step, ())
            return ()

        lax.fori_loop(0, NCH2 // G, _group, ())
        plsc.subcore_barrier()
        for t in range(RPS // K):
            r0 = s * RPS + t * K
            pltpu.sync_copy(acc.at[pl.ds(r0, K)],
                            out_hbm.at[c, pl.ds(r0, K)])

    return scat(hs0, hs1, row_w, col_w)


def _tc_lin1(xp, w1, b1, dega, degb):
    def body(x_ref, w_ref, b_ref, da_ref, db_ref, h_ref, hs0_ref, hs1_ref,
             dis_ref, dis2_ref):
        i = pl.program_id(0)
        row = lax.broadcasted_iota(jnp.int32, (RB, 1), 0) + i * RB
        mask = row < N
        d = da_ref[...] + db_ref[...] + 1.0
        dis = jnp.where(mask, lax.rsqrt(d), 0.0)
        dis2 = jnp.where(mask, 1.0 / d, 0.0)
        dis_ref[...] = dis
        dis2_ref[...] = dis2
        h = jnp.dot(x_ref[...], w_ref[...], preferred_element_type=jnp.float32)
        h = jnp.maximum(h + b_ref[...], 0.0)
        h_ref[...] = h
        hs = h * dis
        hs0_ref[...] = hs[:, :HH]
        hs1_ref[...] = hs[:, HH:]

    return pl.pallas_call(
        body,
        grid=(NP // RB,),
        in_specs=[
            pl.BlockSpec((RB, DIN), lambda i: (i, 0)),
            pl.BlockSpec((DIN, H), lambda i: (0, 0)),
            pl.BlockSpec((1, H), lambda i: (0, 0)),
            pl.BlockSpec((RB, 1), lambda i: (i, 0)),
            pl.BlockSpec((RB, 1), lambda i: (i, 0)),
        ],
        out_specs=[pl.BlockSpec((RB, H), lambda i: (i, 0)),
                   pl.BlockSpec((RB, HH), lambda i: (i, 0)),
                   pl.BlockSpec((RB, HH), lambda i: (i, 0)),
                   pl.BlockSpec((RB, 1), lambda i: (i, 0)),
                   pl.BlockSpec((RB, 1), lambda i: (i, 0))],
        out_shape=[jax.ShapeDtypeStruct((NP, H), jnp.float32),
                   jax.ShapeDtypeStruct((NP, HH), jnp.float32),
                   jax.ShapeDtypeStruct((NP, HH), jnp.float32),
                   jax.ShapeDtypeStruct((NP, 1), jnp.float32),
                   jax.ShapeDtypeStruct((NP, 1), jnp.float32)],
    )(xp, w1, b1.reshape(1, H), dega, degb)


def _tc_layer(q0, q1, h, x0, dis_c, dis2_c, w, beta):
    def body(q0_ref, q1_ref, h_ref, x0_ref, dis_ref, dis2_ref, w_ref,
             hn_ref, hs0_ref, hs1_ref):
        psum = jnp.concatenate([q0_ref[...], q1_ref[...]], axis=1)
        agg = dis_ref[...] * psum + dis2_ref[...] * h_ref[...]
        hh = (1.0 - ALPHA) * agg + ALPHA * x0_ref[...]
        m = jnp.dot(hh, w_ref[...], preferred_element_type=jnp.float32)
        hn = jnp.maximum((1.0 - beta) * hh + beta * m, 0.0)
        hn_ref[...] = hn
        hs = hn * dis_ref[...]
        hs0_ref[...] = hs[:, :HH]
        hs1_ref[...] = hs[:, HH:]

    return pl.pallas_call(
        body,
        grid=(NP // RB,),
        in_specs=[
            pl.BlockSpec((RB, HH), lambda i: (i, 0)),
            pl.BlockSpec((RB, HH), lambda i: (i, 0)),
            pl.BlockSpec((RB, H), lambda i: (i, 0)),
            pl.BlockSpec((RB, H), lambda i: (i, 0)),
            pl.BlockSpec((RB, 1), lambda i: (i, 0)),
            pl.BlockSpec((RB, 1), lambda i: (i, 0)),
            pl.BlockSpec((H, H), lambda i: (0, 0)),
        ],
        out_specs=[pl.BlockSpec((RB, H), lambda i: (i, 0)),
                   pl.BlockSpec((RB, HH), lambda i: (i, 0)),
                   pl.BlockSpec((RB, HH), lambda i: (i, 0))],
        out_shape=[jax.ShapeDtypeStruct((NP, H), jnp.float32),
                   jax.ShapeDtypeStruct((NP, HH), jnp.float32),
                   jax.ShapeDtypeStruct((NP, HH), jnp.float32)],
    )(q0, q1, h, x0, dis_c, dis2_c, w)


def _tc_lin2(h, w2, b2):
    def body(h_ref, w_ref, b_ref, o_ref):
        o_ref[...] = (jnp.dot(h_ref[...], w_ref[...],
                              preferred_element_type=jnp.float32)
                      + b_ref[...])

    return pl.pallas_call(
        body,
        grid=(NP // RB,),
        in_specs=[
            pl.BlockSpec((RB, H), lambda i: (i, 0)),
            pl.BlockSpec((H, DOUT), lambda i: (0, 0)),
            pl.BlockSpec((1, DOUT), lambda i: (0, 0)),
        ],
        out_specs=pl.BlockSpec((RB, DOUT), lambda i: (i, 0)),
        out_shape=jax.ShapeDtypeStruct((NP, DOUT), jnp.float32),
    )(h, w2, b2.reshape(1, DOUT))


def kernel(x, edge_index, w_lin1, b_lin1, conv_w, w_lin2, b_lin2):
    xp = jnp.pad(x, ((0, NP - N), (0, 0)))
    pad = EP - E
    rowp = jnp.concatenate(
        [edge_index[0], jnp.full((pad,), NP - 1, jnp.int32)])
    colp = jnp.concatenate(
        [edge_index[1], jnp.full((pad,), NP - 1, jnp.int32)])
    row_w = rowp.reshape(NS, NCH2, K)
    col_w = colp.reshape(NS, NCH2, K)

    degs = _sc_degree(colp.reshape(NW, NCH, K))
    dega = degs[:NP].reshape(NP, 1)
    degb = degs[NP:].reshape(NP, 1)

    h, hs0, hs1, dis_c, dis2_c = _tc_lin1(xp, w_lin1, b_lin1, dega, degb)
    x0 = h
    for i in range(L):
        beta = math.log(THETA / (i + 1) + 1.0)
        part = _sc_scatter_partials(hs0, hs1, row_w, col_w)
        h, hs0, hs1 = _tc_layer(part[0], part[1], h, x0, dis_c, dis2_c,
                                conv_w[i], beta)

    out = _tc_lin2(h, w_lin2, b_lin2)
    return out[:N]


# speedup vs baseline: 1.1403x; 1.1403x over previous
import functools
import math

import jax
import jax.numpy as jnp
from jax import lax
from jax.experimental import pallas as pl
from jax.experimental.pallas import tpu as pltpu
from jax.experimental.pallas import tpu_sc as plsc

N, DIN, H, DOUT, E, L = 10000, 128, 128, 64, 320000, 8
ALPHA, THETA = 0.1, 0.5

NP = 10240
NC, NS = 2, 16
NW = NC * NS
K = 128
NCH = 80
EP = NW * NCH * K
RPS = NP // NS
RB = 1024


def _sc_mesh():
    return plsc.VectorSubcoreMesh(core_axis_name="c", subcore_axis_name="s")


def _sc_degree(col_w):
    @functools.partial(
        pl.kernel,
        out_type=jax.ShapeDtypeStruct((NC * NP,), jnp.float32),
        mesh=_sc_mesh(),
        scratch_types=[
            pltpu.VMEM((NCH, K), jnp.int32),
            pltpu.VMEM((K,), jnp.float32),
            pltpu.VMEM((RPS,), jnp.float32),
            pltpu.VMEM_SHARED((NP,), jnp.float32),
            pltpu.SemaphoreType.DMA,
        ],
    )
    def degk(col_hbm, out_hbm, col_v, ones_v, z_v, acc, sem):
        c = lax.axis_index("c")
        s = lax.axis_index("s")
        w = c * NS + s
        pltpu.sync_copy(col_hbm.at[w], col_v)
        for b in range(K // 16):
            ones_v[pl.ds(b * 16, 16)] = jnp.ones((16,), jnp.float32)
        for b in range(RPS // 16):
            z_v[pl.ds(b * 16, 16)] = jnp.zeros((16,), jnp.float32)
        pltpu.sync_copy(z_v, acc.at[pl.ds(s * RPS, RPS)])
        plsc.subcore_barrier()

        def _step(j, _):
            pltpu.async_copy(ones_v, acc.at[col_v.at[j]], sem, add=True)
            return ()

        lax.fori_loop(0, NCH, _step, ())
        pltpu.make_async_copy(col_hbm.at[w], col_v, sem).wait()
        plsc.subcore_barrier()
        pltpu.sync_copy(acc.at[pl.ds(s * RPS, RPS)],
                        out_hbm.at[pl.ds(c * NP + s * RPS, RPS)])

    return degk(col_w)


HH = H // 2
NCH2 = EP // (NS * K)
G = 40


def _sc_scatter_partials(hs, row_w, col_w):
    @functools.partial(
        pl.kernel,
        out_type=jax.ShapeDtypeStruct((NP, H), jnp.float32),
        mesh=_sc_mesh(),
        scratch_types=[
            pltpu.VMEM((G, K), jnp.int32),
            pltpu.VMEM((G, K), jnp.int32),
            pltpu.VMEM((2, K, HH), jnp.float32),
            pltpu.VMEM_SHARED((NP, HH), jnp.float32),
            pltpu.VMEM_SHARED((NP, HH), jnp.float32),
            pltpu.SemaphoreType.DMA,
            pltpu.SemaphoreType.DMA,
        ],
        compiler_params=pltpu.CompilerParams(use_tc_tiling_on_sc=False),
    )
    def scat(hs_hbm, row_hbm, col_hbm, out_hbm, row_v, col_v, buf,
             stage, acc, sem0, sem1):
        c = lax.axis_index("c")
        s = lax.axis_index("s")
        co = c * HH

        pltpu.sync_copy(hs_hbm.at[pl.ds(s * RPS, RPS), pl.ds(co, HH)],
                        stage.at[pl.ds(s * RPS, RPS)])

        def _zrow(i, _):
            for b in range(HH // 16):
                buf[0, i, pl.ds(b * 16, 16)] = jnp.zeros((16,), jnp.float32)
            return ()

        lax.fori_loop(0, K, _zrow, ())
        for t in range(RPS // K):
            pltpu.sync_copy(buf.at[0], acc.at[pl.ds(s * RPS + t * K, K)])
        plsc.subcore_barrier()

        def _group(g, _):
            pltpu.sync_copy(row_hbm.at[s, pl.ds(g * G, G)], row_v)
            pltpu.sync_copy(col_hbm.at[s, pl.ds(g * G, G)], col_v)
            pltpu.async_copy(stage.at[row_v.at[0]], buf.at[0], sem0)
            pltpu.async_copy(stage.at[row_v.at[1]], buf.at[1], sem1)

            def _step(k2, _):
                j = 2 * k2
                for b, sem in ((0, sem0), (1, sem1)):
                    jj = j + b
                    pltpu.make_async_copy(stage.at[row_v.at[jj]], buf.at[b],
                                          sem).wait()
                    pltpu.sync_copy(buf.at[b], acc.at[col_v.at[jj]], add=True)

                    @pl.when(jj + 2 < G)
                    def _():
                        pltpu.async_copy(stage.at[row_v.at[jj + 2]],
                                         buf.at[b], sem)
                return ()

            lax.fori_loop(0, G // 2, _step, ())
            return ()

        lax.fori_loop(0, NCH2 // G, _group, ())
        plsc.subcore_barrier()
        for t in range(RPS // K):
            r0 = s * RPS + t * K
            pltpu.sync_copy(acc.at[pl.ds(r0, K)],
                            out_hbm.at[pl.ds(r0, K), pl.ds(co, HH)])

    return scat(hs, row_w, col_w)


def _tc_lin1(xp, w1, b1, dega, degb):
    def body(x_ref, w_ref, b_ref, da_ref, db_ref, h_ref, hs_ref,
             dis_ref, dis2_ref):
        i = pl.program_id(0)
        row = lax.broadcasted_iota(jnp.int32, (RB, 1), 0) + i * RB
        mask = row < N
        d = da_ref[...] + db_ref[...] + 1.0
        dis = jnp.where(mask, lax.rsqrt(d), 0.0)
        dis2 = jnp.where(mask, 1.0 / d, 0.0)
        dis_ref[...] = dis
        dis2_ref[...] = dis2
        h = jnp.dot(x_ref[...], w_ref[...], preferred_element_type=jnp.float32)
        h = jnp.maximum(h + b_ref[...], 0.0)
        h_ref[...] = h
        hs_ref[...] = h * dis

    return pl.pallas_call(
        body,
        grid=(NP // RB,),
        in_specs=[
            pl.BlockSpec((RB, DIN), lambda i: (i, 0)),
            pl.BlockSpec((DIN, H), lambda i: (0, 0)),
            pl.BlockSpec((1, H), lambda i: (0, 0)),
            pl.BlockSpec((RB, 1), lambda i: (i, 0)),
            pl.BlockSpec((RB, 1), lambda i: (i, 0)),
        ],
        out_specs=[pl.BlockSpec((RB, H), lambda i: (i, 0)),
                   pl.BlockSpec((RB, H), lambda i: (i, 0)),
                   pl.BlockSpec((RB, 1), lambda i: (i, 0)),
                   pl.BlockSpec((RB, 1), lambda i: (i, 0))],
        out_shape=[jax.ShapeDtypeStruct((NP, H), jnp.float32),
                   jax.ShapeDtypeStruct((NP, H), jnp.float32),
                   jax.ShapeDtypeStruct((NP, 1), jnp.float32),
                   jax.ShapeDtypeStruct((NP, 1), jnp.float32)],
    )(xp, w1, b1.reshape(1, H), dega, degb)


def _tc_layer(psum, h, x0, dis_c, dis2_c, w, beta):
    def body(p_ref, h_ref, x0_ref, dis_ref, dis2_ref, w_ref,
             hn_ref, hs_ref):
        agg = dis_ref[...] * p_ref[...] + dis2_ref[...] * h_ref[...]
        hh = (1.0 - ALPHA) * agg + ALPHA * x0_ref[...]
        m = jnp.dot(hh, w_ref[...], preferred_element_type=jnp.float32)
        hn = jnp.maximum((1.0 - beta) * hh + beta * m, 0.0)
        hn_ref[...] = hn
        hs_ref[...] = hn * dis_ref[...]

    return pl.pallas_call(
        body,
        grid=(NP // RB,),
        in_specs=[
            pl.BlockSpec((RB, H), lambda i: (i, 0)),
            pl.BlockSpec((RB, H), lambda i: (i, 0)),
            pl.BlockSpec((RB, H), lambda i: (i, 0)),
            pl.BlockSpec((RB, 1), lambda i: (i, 0)),
            pl.BlockSpec((RB, 1), lambda i: (i, 0)),
            pl.BlockSpec((H, H), lambda i: (0, 0)),
        ],
        out_specs=[pl.BlockSpec((RB, H), lambda i: (i, 0)),
                   pl.BlockSpec((RB, H), lambda i: (i, 0))],
        out_shape=[jax.ShapeDtypeStruct((NP, H), jnp.float32),
                   jax.ShapeDtypeStruct((NP, H), jnp.float32)],
    )(psum, h, x0, dis_c, dis2_c, w)


def _tc_lin2(h, w2, b2):
    def body(h_ref, w_ref, b_ref, o_ref):
        o_ref[...] = (jnp.dot(h_ref[...], w_ref[...],
                              preferred_element_type=jnp.float32)
                      + b_ref[...])

    return pl.pallas_call(
        body,
        grid=(NP // RB,),
        in_specs=[
            pl.BlockSpec((RB, H), lambda i: (i, 0)),
            pl.BlockSpec((H, DOUT), lambda i: (0, 0)),
            pl.BlockSpec((1, DOUT), lambda i: (0, 0)),
        ],
        out_specs=pl.BlockSpec((RB, DOUT), lambda i: (i, 0)),
        out_shape=jax.ShapeDtypeStruct((NP, DOUT), jnp.float32),
    )(h, w2, b2.reshape(1, DOUT))


def kernel(x, edge_index, w_lin1, b_lin1, conv_w, w_lin2, b_lin2):
    xp = jnp.pad(x, ((0, NP - N), (0, 0)))
    pad = EP - E
    rowp = jnp.concatenate(
        [edge_index[0], jnp.full((pad,), NP - 1, jnp.int32)])
    colp = jnp.concatenate(
        [edge_index[1], jnp.full((pad,), NP - 1, jnp.int32)])
    row_w = rowp.reshape(NS, NCH2, K)
    col_w = colp.reshape(NS, NCH2, K)

    degs = _sc_degree(colp.reshape(NW, NCH, K))
    dega = degs[:NP].reshape(NP, 1)
    degb = degs[NP:].reshape(NP, 1)

    h, hs, dis_c, dis2_c = _tc_lin1(xp, w_lin1, b_lin1, dega, degb)
    x0 = h
    for i in range(L):
        beta = math.log(THETA / (i + 1) + 1.0)
        psum = _sc_scatter_partials(hs, row_w, col_w)
        h, hs = _tc_layer(psum, h, x0, dis_c, dis2_c, conv_w[i], beta)

    out = _tc_lin2(h, w_lin2, b_lin2)
    return out[:N]


# speedup vs baseline: 1.2690x; 1.1128x over previous
import functools
import math

import jax
import jax.numpy as jnp
from jax import lax
from jax.experimental import pallas as pl
from jax.experimental.pallas import tpu as pltpu
from jax.experimental.pallas import tpu_sc as plsc

N, DIN, H, DOUT, E, L = 10000, 128, 128, 64, 320000, 8
ALPHA, THETA = 0.1, 0.5

NP = 10240
NC, NS = 2, 16
NW = NC * NS
K = 128
NCH = 80
EP = NW * NCH * K
RPS = NP // NS
RB = 1024


def _sc_mesh():
    return plsc.VectorSubcoreMesh(core_axis_name="c", subcore_axis_name="s")


def _sc_degree(col_w):
    @functools.partial(
        pl.kernel,
        out_type=jax.ShapeDtypeStruct((NC * NP,), jnp.float32),
        mesh=_sc_mesh(),
        scratch_types=[
            pltpu.VMEM((NCH, K), jnp.int32),
            pltpu.VMEM((K,), jnp.float32),
            pltpu.VMEM((RPS,), jnp.float32),
            pltpu.VMEM_SHARED((NP,), jnp.float32),
            pltpu.SemaphoreType.DMA,
        ],
    )
    def degk(col_hbm, out_hbm, col_v, ones_v, z_v, acc, sem):
        c = lax.axis_index("c")
        s = lax.axis_index("s")
        w = c * NS + s
        pltpu.sync_copy(col_hbm.at[w], col_v)
        for b in range(K // 16):
            ones_v[pl.ds(b * 16, 16)] = jnp.ones((16,), jnp.float32)
        for b in range(RPS // 16):
            z_v[pl.ds(b * 16, 16)] = jnp.zeros((16,), jnp.float32)
        pltpu.sync_copy(z_v, acc.at[pl.ds(s * RPS, RPS)])
        plsc.subcore_barrier()

        def _step(j, _):
            pltpu.async_copy(ones_v, acc.at[col_v.at[j]], sem, add=True)
            return ()

        lax.fori_loop(0, NCH, _step, ())
        pltpu.make_async_copy(col_hbm.at[w], col_v, sem).wait()
        plsc.subcore_barrier()
        pltpu.sync_copy(acc.at[pl.ds(s * RPS, RPS)],
                        out_hbm.at[pl.ds(c * NP + s * RPS, RPS)])

    return degk(col_w)


HH = H // 2
NCH2 = EP // (NS * K)
G = 20
NB = 4


def _sc_scatter_partials(hs, row_w, col_w):
    @functools.partial(
        pl.kernel,
        out_type=jax.ShapeDtypeStruct((NP, H), jnp.float32),
        mesh=_sc_mesh(),
        scratch_types=[
            pltpu.VMEM((G, K), jnp.int32),
            pltpu.VMEM((G, K), jnp.int32),
            pltpu.VMEM((NB, K, HH), jnp.float32),
            pltpu.VMEM_SHARED((NP, HH), jnp.float32),
            pltpu.VMEM_SHARED((NP, HH), jnp.float32),
            [pltpu.SemaphoreType.DMA] * NB,
            [pltpu.SemaphoreType.DMA] * NB,
        ],
        compiler_params=pltpu.CompilerParams(use_tc_tiling_on_sc=False),
    )
    def scat(hs_hbm, row_hbm, col_hbm, out_hbm, row_v, col_v, buf,
             stage, acc, gsem, ssem):
        c = lax.axis_index("c")
        s = lax.axis_index("s")
        co = c * HH

        pltpu.sync_copy(hs_hbm.at[pl.ds(s * RPS, RPS), pl.ds(co, HH)],
                        stage.at[pl.ds(s * RPS, RPS)])

        def _zrow(i, _):
            for b in range(HH // 16):
                buf[0, i, pl.ds(b * 16, 16)] = jnp.zeros((16,), jnp.float32)
            return ()

        lax.fori_loop(0, K, _zrow, ())
        for t in range(RPS // K):
            pltpu.sync_copy(buf.at[0], acc.at[pl.ds(s * RPS + t * K, K)])
        plsc.subcore_barrier()

        def _group(g, _):
            pltpu.sync_copy(row_hbm.at[s, pl.ds(g * G, G)], row_v)
            pltpu.sync_copy(col_hbm.at[s, pl.ds(g * G, G)], col_v)
            pltpu.async_copy(stage.at[row_v.at[0]], buf.at[0], gsem[0])
            pltpu.async_copy(stage.at[row_v.at[1]], buf.at[1], gsem[1])

            def _step(k2, _):
                j0 = NB * k2
                for b in range(NB):
                    j = j0 + b
                    pltpu.make_async_copy(stage.at[row_v.at[j]], buf.at[b],
                                          gsem[b]).wait()
                    pltpu.async_copy(buf.at[b], acc.at[col_v.at[j]],
                                     ssem[b], add=True)
                    m = j + 2
                    bm = (b + 2) % NB

                    @pl.when(m < G)
                    def _():
                        @pl.when(m >= NB)
                        def _():
                            pltpu.make_async_copy(
                                buf.at[bm], acc.at[col_v.at[0]],
                                ssem[bm]).wait()

                        pltpu.async_copy(stage.at[row_v.at[m]], buf.at[bm],
                                         gsem[bm])
                return ()

            lax.fori_loop(0, G // NB, _step, ())
            for b in range(NB):
                pltpu.make_async_copy(buf.at[b], acc.at[col_v.at[0]],
                                      ssem[b]).wait()
            return ()

        lax.fori_loop(0, NCH2 // G, _group, ())
        plsc.subcore_barrier()
        for t in range(RPS // K):
            r0 = s * RPS + t * K
            pltpu.sync_copy(acc.at[pl.ds(r0, K)],
                            out_hbm.at[pl.ds(r0, K), pl.ds(co, HH)])

    return scat(hs, row_w, col_w)


def _tc_lin1(xp, w1, b1, dega, degb):
    def body(x_ref, w_ref, b_ref, da_ref, db_ref, h_ref, hs_ref,
             dis_ref, dis2_ref):
        i = pl.program_id(0)
        row = lax.broadcasted_iota(jnp.int32, (RB, 1), 0) + i * RB
        mask = row < N
        d = da_ref[...] + db_ref[...] + 1.0
        dis = jnp.where(mask, lax.rsqrt(d), 0.0)
        dis2 = jnp.where(mask, 1.0 / d, 0.0)
        dis_ref[...] = dis
        dis2_ref[...] = dis2
        h = jnp.dot(x_ref[...], w_ref[...], preferred_element_type=jnp.float32)
        h = jnp.maximum(h + b_ref[...], 0.0)
        h_ref[...] = h
        hs_ref[...] = h * dis

    return pl.pallas_call(
        body,
        grid=(NP // RB,),
        in_specs=[
            pl.BlockSpec((RB, DIN), lambda i: (i, 0)),
            pl.BlockSpec((DIN, H), lambda i: (0, 0)),
            pl.BlockSpec((1, H), lambda i: (0, 0)),
            pl.BlockSpec((RB, 1), lambda i: (i, 0)),
            pl.BlockSpec((RB, 1), lambda i: (i, 0)),
        ],
        out_specs=[pl.BlockSpec((RB, H), lambda i: (i, 0)),
                   pl.BlockSpec((RB, H), lambda i: (i, 0)),
                   pl.BlockSpec((RB, 1), lambda i: (i, 0)),
                   pl.BlockSpec((RB, 1), lambda i: (i, 0))],
        out_shape=[jax.ShapeDtypeStruct((NP, H), jnp.float32),
                   jax.ShapeDtypeStruct((NP, H), jnp.float32),
                   jax.ShapeDtypeStruct((NP, 1), jnp.float32),
                   jax.ShapeDtypeStruct((NP, 1), jnp.float32)],
    )(xp, w1, b1.reshape(1, H), dega, degb)


def _tc_layer(psum, h, x0, dis_c, dis2_c, w, beta):
    def body(p_ref, h_ref, x0_ref, dis_ref, dis2_ref, w_ref,
             hn_ref, hs_ref):
        agg = dis_ref[...] * p_ref[...] + dis2_ref[...] * h_ref[...]
        hh = (1.0 - ALPHA) * agg + ALPHA * x0_ref[...]
        m = jnp.dot(hh, w_ref[...], preferred_element_type=jnp.float32)
        hn = jnp.maximum((1.0 - beta) * hh + beta * m, 0.0)
        hn_ref[...] = hn
        hs_ref[...] = hn * dis_ref[...]

    return pl.pallas_call(
        body,
        grid=(NP // RB,),
        in_specs=[
            pl.BlockSpec((RB, H), lambda i: (i, 0)),
            pl.BlockSpec((RB, H), lambda i: (i, 0)),
            pl.BlockSpec((RB, H), lambda i: (i, 0)),
            pl.BlockSpec((RB, 1), lambda i: (i, 0)),
            pl.BlockSpec((RB, 1), lambda i: (i, 0)),
            pl.BlockSpec((H, H), lambda i: (0, 0)),
        ],
        out_specs=[pl.BlockSpec((RB, H), lambda i: (i, 0)),
                   pl.BlockSpec((RB, H), lambda i: (i, 0))],
        out_shape=[jax.ShapeDtypeStruct((NP, H), jnp.float32),
                   jax.ShapeDtypeStruct((NP, H), jnp.float32)],
    )(psum, h, x0, dis_c, dis2_c, w)


def _tc_lin2(h, w2, b2):
    def body(h_ref, w_ref, b_ref, o_ref):
        o_ref[...] = (jnp.dot(h_ref[...], w_ref[...],
                              preferred_element_type=jnp.float32)
                      + b_ref[...])

    return pl.pallas_call(
        body,
        grid=(NP // RB,),
        in_specs=[
            pl.BlockSpec((RB, H), lambda i: (i, 0)),
            pl.BlockSpec((H, DOUT), lambda i: (0, 0)),
            pl.BlockSpec((1, DOUT), lambda i: (0, 0)),
        ],
        out_specs=pl.BlockSpec((RB, DOUT), lambda i: (i, 0)),
        out_shape=jax.ShapeDtypeStruct((NP, DOUT), jnp.float32),
    )(h, w2, b2.reshape(1, DOUT))


def kernel(x, edge_index, w_lin1, b_lin1, conv_w, w_lin2, b_lin2):
    xp = jnp.pad(x, ((0, NP - N), (0, 0)))
    pad = EP - E
    rowp = jnp.concatenate(
        [edge_index[0], jnp.full((pad,), NP - 1, jnp.int32)])
    colp = jnp.concatenate(
        [edge_index[1], jnp.full((pad,), NP - 1, jnp.int32)])
    row_w = rowp.reshape(NS, NCH2, K)
    col_w = colp.reshape(NS, NCH2, K)

    degs = _sc_degree(colp.reshape(NW, NCH, K))
    dega = degs[:NP].reshape(NP, 1)
    degb = degs[NP:].reshape(NP, 1)

    h, hs, dis_c, dis2_c = _tc_lin1(xp, w_lin1, b_lin1, dega, degb)
    x0 = h
    for i in range(L):
        beta = math.log(THETA / (i + 1) + 1.0)
        psum = _sc_scatter_partials(hs, row_w, col_w)
        h, hs = _tc_layer(psum, h, x0, dis_c, dis2_c, conv_w[i], beta)

    out = _tc_lin2(h, w_lin2, b_lin2)
    return out[:N]


# speedup vs baseline: 1.3362x; 1.0529x over previous
import functools
import math

import jax
import jax.numpy as jnp
from jax import lax
from jax.experimental import pallas as pl
from jax.experimental.pallas import tpu as pltpu
from jax.experimental.pallas import tpu_sc as plsc

N, DIN, H, DOUT, E, L = 10000, 128, 128, 64, 320000, 8
ALPHA, THETA = 0.1, 0.5

NP = 10240
NC, NS = 2, 16
NW = NC * NS
K = 128
NCH = 80
EP = NW * NCH * K
RPS = NP // NS
RB = 2048


def _sc_mesh():
    return plsc.VectorSubcoreMesh(core_axis_name="c", subcore_axis_name="s")


def _sc_degree(col_w):
    @functools.partial(
        pl.kernel,
        out_type=jax.ShapeDtypeStruct((NC * NP,), jnp.float32),
        mesh=_sc_mesh(),
        scratch_types=[
            pltpu.VMEM((NCH, K), jnp.int32),
            pltpu.VMEM((K,), jnp.float32),
            pltpu.VMEM((RPS,), jnp.float32),
            pltpu.VMEM_SHARED((NP,), jnp.float32),
            pltpu.SemaphoreType.DMA,
        ],
    )
    def degk(col_hbm, out_hbm, col_v, ones_v, z_v, acc, sem):
        c = lax.axis_index("c")
        s = lax.axis_index("s")
        w = c * NS + s
        pltpu.sync_copy(col_hbm.at[w], col_v)
        for b in range(K // 16):
            ones_v[pl.ds(b * 16, 16)] = jnp.ones((16,), jnp.float32)
        for b in range(RPS // 16):
            z_v[pl.ds(b * 16, 16)] = jnp.zeros((16,), jnp.float32)
        pltpu.sync_copy(z_v, acc.at[pl.ds(s * RPS, RPS)])
        plsc.subcore_barrier()

        def _step(j, _):
            pltpu.async_copy(ones_v, acc.at[col_v.at[j]], sem, add=True)
            return ()

        lax.fori_loop(0, NCH, _step, ())
        pltpu.make_async_copy(col_hbm.at[w], col_v, sem).wait()
        plsc.subcore_barrier()
        pltpu.sync_copy(acc.at[pl.ds(s * RPS, RPS)],
                        out_hbm.at[pl.ds(c * NP + s * RPS, RPS)])

    return degk(col_w)


HH = H // 2
NCH2 = EP // (NS * K)
G = 40
NB = 4


def _sc_scatter_partials(hs, row_w, col_w):
    @functools.partial(
        pl.kernel,
        out_type=jax.ShapeDtypeStruct((NP, H), jnp.float32),
        mesh=_sc_mesh(),
        scratch_types=[
            pltpu.VMEM((G, K), jnp.int32),
            pltpu.VMEM((G, K), jnp.int32),
            pltpu.VMEM((NB, K, HH), jnp.float32),
            pltpu.VMEM_SHARED((NP, HH), jnp.float32),
            pltpu.VMEM_SHARED((NP, HH), jnp.float32),
            [pltpu.SemaphoreType.DMA] * NB,
            [pltpu.SemaphoreType.DMA] * NB,
        ],
        compiler_params=pltpu.CompilerParams(use_tc_tiling_on_sc=False),
    )
    def scat(hs_hbm, row_hbm, col_hbm, out_hbm, row_v, col_v, buf,
             stage, acc, gsem, ssem):
        c = lax.axis_index("c")
        s = lax.axis_index("s")
        co = c * HH

        pltpu.sync_copy(hs_hbm.at[pl.ds(s * RPS, RPS), pl.ds(co, HH)],
                        stage.at[pl.ds(s * RPS, RPS)])

        def _zrow(i, _):
            for b in range(HH // 16):
                buf[0, i, pl.ds(b * 16, 16)] = jnp.zeros((16,), jnp.float32)
            return ()

        lax.fori_loop(0, K, _zrow, ())
        for t in range(RPS // K):
            pltpu.sync_copy(buf.at[0], acc.at[pl.ds(s * RPS + t * K, K)])
        plsc.subcore_barrier()

        def _group(g, _):
            pltpu.sync_copy(row_hbm.at[s, pl.ds(g * G, G)], row_v)
            pltpu.sync_copy(col_hbm.at[s, pl.ds(g * G, G)], col_v)
            pltpu.async_copy(stage.at[row_v.at[0]], buf.at[0], gsem[0])
            pltpu.async_copy(stage.at[row_v.at[1]], buf.at[1], gsem[1])

            def _step(k2, _):
                j0 = NB * k2
                for b in range(NB):
                    j = j0 + b
                    pltpu.make_async_copy(stage.at[row_v.at[j]], buf.at[b],
                                          gsem[b]).wait()
                    pltpu.async_copy(buf.at[b], acc.at[col_v.at[j]],
                                     ssem[b], add=True)
                    m = j + 2
                    bm = (b + 2) % NB

                    @pl.when(m < G)
                    def _():
                        @pl.when(m >= NB)
                        def _():
                            pltpu.make_async_copy(
                                buf.at[bm], acc.at[col_v.at[0]],
                                ssem[bm]).wait()

                        pltpu.async_copy(stage.at[row_v.at[m]], buf.at[bm],
                                         gsem[bm])
                return ()

            lax.fori_loop(0, G // NB, _step, ())
            for b in range(NB):
                pltpu.make_async_copy(buf.at[b], acc.at[col_v.at[0]],
                                      ssem[b]).wait()
            return ()

        lax.fori_loop(0, NCH2 // G, _group, ())
        plsc.subcore_barrier()
        for t in range(RPS // K):
            r0 = s * RPS + t * K
            pltpu.sync_copy(acc.at[pl.ds(r0, K)],
                            out_hbm.at[pl.ds(r0, K), pl.ds(co, HH)])

    return scat(hs, row_w, col_w)


def _tc_lin1(xp, w1, b1, dega, degb):
    def body(x_ref, w_ref, b_ref, da_ref, db_ref, h_ref, hs_ref,
             dis_ref, dis2_ref):
        i = pl.program_id(0)
        row = lax.broadcasted_iota(jnp.int32, (RB, 1), 0) + i * RB
        mask = row < N
        d = da_ref[...] + db_ref[...] + 1.0
        dis = jnp.where(mask, lax.rsqrt(d), 0.0)
        dis2 = jnp.where(mask, 1.0 / d, 0.0)
        dis_ref[...] = dis
        dis2_ref[...] = dis2
        h = jnp.dot(x_ref[...], w_ref[...], preferred_element_type=jnp.float32)
        h = jnp.maximum(h + b_ref[...], 0.0)
        h_ref[...] = h
        hs_ref[...] = h * dis

    return pl.pallas_call(
        body,
        grid=(NP // RB,),
        in_specs=[
            pl.BlockSpec((RB, DIN), lambda i: (i, 0)),
            pl.BlockSpec((DIN, H), lambda i: (0, 0)),
            pl.BlockSpec((1, H), lambda i: (0, 0)),
            pl.BlockSpec((RB, 1), lambda i: (i, 0)),
            pl.BlockSpec((RB, 1), lambda i: (i, 0)),
        ],
        out_specs=[pl.BlockSpec((RB, H), lambda i: (i, 0)),
                   pl.BlockSpec((RB, H), lambda i: (i, 0)),
                   pl.BlockSpec((RB, 1), lambda i: (i, 0)),
                   pl.BlockSpec((RB, 1), lambda i: (i, 0))],
        out_shape=[jax.ShapeDtypeStruct((NP, H), jnp.float32),
                   jax.ShapeDtypeStruct((NP, H), jnp.float32),
                   jax.ShapeDtypeStruct((NP, 1), jnp.float32),
                   jax.ShapeDtypeStruct((NP, 1), jnp.float32)],
    )(xp, w1, b1.reshape(1, H), dega, degb)


def _tc_layer(psum, h, x0, dis_c, dis2_c, w, beta):
    def body(p_ref, h_ref, x0_ref, dis_ref, dis2_ref, w_ref,
             hn_ref, hs_ref):
        agg = dis_ref[...] * p_ref[...] + dis2_ref[...] * h_ref[...]
        hh = (1.0 - ALPHA) * agg + ALPHA * x0_ref[...]
        m = jnp.dot(hh, w_ref[...], preferred_element_type=jnp.float32)
        hn = jnp.maximum((1.0 - beta) * hh + beta * m, 0.0)
        hn_ref[...] = hn
        hs_ref[...] = hn * dis_ref[...]

    return pl.pallas_call(
        body,
        grid=(NP // RB,),
        in_specs=[
            pl.BlockSpec((RB, H), lambda i: (i, 0)),
            pl.BlockSpec((RB, H), lambda i: (i, 0)),
            pl.BlockSpec((RB, H), lambda i: (i, 0)),
            pl.BlockSpec((RB, 1), lambda i: (i, 0)),
            pl.BlockSpec((RB, 1), lambda i: (i, 0)),
            pl.BlockSpec((H, H), lambda i: (0, 0)),
        ],
        out_specs=[pl.BlockSpec((RB, H), lambda i: (i, 0)),
                   pl.BlockSpec((RB, H), lambda i: (i, 0))],
        out_shape=[jax.ShapeDtypeStruct((NP, H), jnp.float32),
                   jax.ShapeDtypeStruct((NP, H), jnp.float32)],
    )(psum, h, x0, dis_c, dis2_c, w)


def _tc_lin2(h, w2, b2):
    def body(h_ref, w_ref, b_ref, o_ref):
        o_ref[...] = (jnp.dot(h_ref[...], w_ref[...],
                              preferred_element_type=jnp.float32)
                      + b_ref[...])

    return pl.pallas_call(
        body,
        grid=(NP // RB,),
        in_specs=[
            pl.BlockSpec((RB, H), lambda i: (i, 0)),
            pl.BlockSpec((H, DOUT), lambda i: (0, 0)),
            pl.BlockSpec((1, DOUT), lambda i: (0, 0)),
        ],
        out_specs=pl.BlockSpec((RB, DOUT), lambda i: (i, 0)),
        out_shape=jax.ShapeDtypeStruct((NP, DOUT), jnp.float32),
    )(h, w2, b2.reshape(1, DOUT))


def kernel(x, edge_index, w_lin1, b_lin1, conv_w, w_lin2, b_lin2):
    xp = jnp.pad(x, ((0, NP - N), (0, 0)))
    pad = EP - E
    rowp = jnp.concatenate(
        [edge_index[0], jnp.full((pad,), NP - 1, jnp.int32)])
    colp = jnp.concatenate(
        [edge_index[1], jnp.full((pad,), NP - 1, jnp.int32)])
    row_w = rowp.reshape(NS, NCH2, K)
    col_w = colp.reshape(NS, NCH2, K)

    degs = _sc_degree(colp.reshape(NW, NCH, K))
    dega = degs[:NP].reshape(NP, 1)
    degb = degs[NP:].reshape(NP, 1)

    h, hs, dis_c, dis2_c = _tc_lin1(xp, w_lin1, b_lin1, dega, degb)
    x0 = h
    for i in range(L):
        beta = math.log(THETA / (i + 1) + 1.0)
        psum = _sc_scatter_partials(hs, row_w, col_w)
        h, hs = _tc_layer(psum, h, x0, dis_c, dis2_c, conv_w[i], beta)

    out = _tc_lin2(h, w_lin2, b_lin2)
    return out[:N]


# speedup vs baseline: 1.3839x; 1.0357x over previous
import functools
import math

import jax
import jax.numpy as jnp
from jax import lax
from jax.experimental import pallas as pl
from jax.experimental.pallas import tpu as pltpu
from jax.experimental.pallas import tpu_sc as plsc

N, DIN, H, DOUT, E, L = 10000, 128, 128, 64, 320000, 8
ALPHA, THETA = 0.1, 0.5

NP = 10240
NC, NS = 2, 16
NW = NC * NS
K = 128
NCH = 80
EP = NW * NCH * K
RPS = NP // NS
RB = 2048


def _sc_mesh():
    return plsc.VectorSubcoreMesh(core_axis_name="c", subcore_axis_name="s")


def _sc_degree(col_w):
    @functools.partial(
        pl.kernel,
        out_type=jax.ShapeDtypeStruct((NC * NP,), jnp.float32),
        mesh=_sc_mesh(),
        scratch_types=[
            pltpu.VMEM((NCH, K), jnp.int32),
            pltpu.VMEM((K,), jnp.float32),
            pltpu.VMEM((RPS,), jnp.float32),
            pltpu.VMEM_SHARED((NP,), jnp.float32),
            pltpu.SemaphoreType.DMA,
        ],
    )
    def degk(col_hbm, out_hbm, col_v, ones_v, z_v, acc, sem):
        c = lax.axis_index("c")
        s = lax.axis_index("s")
        w = c * NS + s
        pltpu.sync_copy(col_hbm.at[w], col_v)
        for b in range(K // 16):
            ones_v[pl.ds(b * 16, 16)] = jnp.ones((16,), jnp.float32)
        for b in range(RPS // 16):
            z_v[pl.ds(b * 16, 16)] = jnp.zeros((16,), jnp.float32)
        pltpu.sync_copy(z_v, acc.at[pl.ds(s * RPS, RPS)])
        plsc.subcore_barrier()

        def _step(j, _):
            pltpu.async_copy(ones_v, acc.at[col_v.at[j]], sem, add=True)
            return ()

        lax.fori_loop(0, NCH, _step, ())
        pltpu.make_async_copy(col_hbm.at[w], col_v, sem).wait()
        plsc.subcore_barrier()
        pltpu.sync_copy(acc.at[pl.ds(s * RPS, RPS)],
                        out_hbm.at[pl.ds(c * NP + s * RPS, RPS)])

    return degk(col_w)


HH = H // 2
NCH2 = EP // (NS * K)
G = 20
NG = 8
NB = 4


def _sc_scatter_partials(hs, row_w, col_w):
    @functools.partial(
        pl.kernel,
        out_type=jax.ShapeDtypeStruct((NP, H), jnp.float32),
        mesh=_sc_mesh(),
        scratch_types=[
            pltpu.VMEM((2, G, K), jnp.int32),
            pltpu.VMEM((2, G, K), jnp.int32),
            pltpu.VMEM((NB, K, HH), jnp.float32),
            pltpu.VMEM_SHARED((NP, HH), jnp.float32),
            pltpu.VMEM_SHARED((NP, HH), jnp.float32),
            [pltpu.SemaphoreType.DMA] * NB,
            [pltpu.SemaphoreType.DMA] * NB,
            pltpu.SemaphoreType.DMA,
        ],
        compiler_params=pltpu.CompilerParams(use_tc_tiling_on_sc=False),
    )
    def scat(hs_hbm, row_hbm, col_hbm, out_hbm, row_v, col_v, buf,
             stage, acc, gsem, ssem, isem):
        c = lax.axis_index("c")
        s = lax.axis_index("s")
        co = c * HH

        pltpu.sync_copy(hs_hbm.at[pl.ds(s * RPS, RPS), pl.ds(co, HH)],
                        stage.at[pl.ds(s * RPS, RPS)])

        def _zrow(i, _):
            for b in range(HH // 16):
                buf[0, i, pl.ds(b * 16, 16)] = jnp.zeros((16,), jnp.float32)
            return ()

        lax.fori_loop(0, K, _zrow, ())
        for t in range(RPS // K):
            pltpu.sync_copy(buf.at[0], acc.at[pl.ds(s * RPS + t * K, K)])
        plsc.subcore_barrier()

        pltpu.sync_copy(row_hbm.at[s, pl.ds(0, G)], row_v.at[0])
        pltpu.sync_copy(col_hbm.at[s, pl.ds(0, G)], col_v.at[0])
        pltpu.async_copy(row_hbm.at[s, pl.ds(G, G)], row_v.at[1], isem)
        pltpu.async_copy(col_hbm.at[s, pl.ds(G, G)], col_v.at[1], isem)

        def _ridx(j):
            return row_v.at[(j // G) % 2, j % G]

        def _cidx(j):
            return col_v.at[(j // G) % 2, j % G]

        pltpu.async_copy(stage.at[_ridx(0)], buf.at[0], gsem[0])
        pltpu.async_copy(stage.at[_ridx(1)], buf.at[1], gsem[1])

        def _step(k2, _):
            j0 = NB * k2
            for b in range(NB):
                j = j0 + b
                pltpu.make_async_copy(stage.at[_ridx(j)], buf.at[b],
                                      gsem[b]).wait()
                pltpu.async_copy(buf.at[b], acc.at[_cidx(j)],
                                 ssem[b], add=True)

                @pl.when(jnp.logical_and(j % G == 4,
                                         jnp.logical_and(j > G,
                                                         j < NCH2 - G)))
                def _():
                    g2 = j // G + 1
                    pltpu.make_async_copy(row_hbm.at[s, pl.ds(0, G)],
                                          row_v.at[0], isem).wait()
                    pltpu.make_async_copy(col_hbm.at[s, pl.ds(0, G)],
                                          col_v.at[0], isem).wait()
                    pltpu.async_copy(row_hbm.at[s, pl.ds(g2 * G, G)],
                                     row_v.at[g2 % 2], isem)
                    pltpu.async_copy(col_hbm.at[s, pl.ds(g2 * G, G)],
                                     col_v.at[g2 % 2], isem)

                m = j + 2
                bm = (b + 2) % NB

                @pl.when(m < NCH2)
                def _():
                    @pl.when(m >= NB)
                    def _():
                        pltpu.make_async_copy(
                            buf.at[bm], acc.at[_cidx(0)],
                            ssem[bm]).wait()

                    pltpu.async_copy(stage.at[_ridx(m)], buf.at[bm],
                                     gsem[bm])
            return ()

        lax.fori_loop(0, NCH2 // NB, _step, ())
        for b in range(NB):
            pltpu.make_async_copy(buf.at[b], acc.at[_cidx(0)],
                                  ssem[b]).wait()
        pltpu.make_async_copy(row_hbm.at[s, pl.ds(0, G)], row_v.at[0],
                              isem).wait()
        pltpu.make_async_copy(col_hbm.at[s, pl.ds(0, G)], col_v.at[0],
                              isem).wait()
        plsc.subcore_barrier()
        for t in range(RPS // K):
            r0 = s * RPS + t * K
            pltpu.sync_copy(acc.at[pl.ds(r0, K)],
                            out_hbm.at[pl.ds(r0, K), pl.ds(co, HH)])

    return scat(hs, row_w, col_w)


def _tc_lin1(xp, w1, b1, dega, degb):
    def body(x_ref, w_ref, b_ref, da_ref, db_ref, h_ref, hs_ref,
             dis_ref, dis2_ref):
        i = pl.program_id(0)
        row = lax.broadcasted_iota(jnp.int32, (RB, 1), 0) + i * RB
        mask = row < N
        d = da_ref[...] + db_ref[...] + 1.0
        dis = jnp.where(mask, lax.rsqrt(d), 0.0)
        dis2 = jnp.where(mask, 1.0 / d, 0.0)
        dis_ref[...] = dis
        dis2_ref[...] = dis2
        h = jnp.dot(x_ref[...], w_ref[...], preferred_element_type=jnp.float32)
        h = jnp.maximum(h + b_ref[...], 0.0)
        h_ref[...] = h
        hs_ref[...] = h * dis

    return pl.pallas_call(
        body,
        grid=(NP // RB,),
        in_specs=[
            pl.BlockSpec((RB, DIN), lambda i: (i, 0)),
            pl.BlockSpec((DIN, H), lambda i: (0, 0)),
            pl.BlockSpec((1, H), lambda i: (0, 0)),
            pl.BlockSpec((RB, 1), lambda i: (i, 0)),
            pl.BlockSpec((RB, 1), lambda i: (i, 0)),
        ],
        out_specs=[pl.BlockSpec((RB, H), lambda i: (i, 0)),
                   pl.BlockSpec((RB, H), lambda i: (i, 0)),
                   pl.BlockSpec((RB, 1), lambda i: (i, 0)),
                   pl.BlockSpec((RB, 1), lambda i: (i, 0))],
        out_shape=[jax.ShapeDtypeStruct((NP, H), jnp.float32),
                   jax.ShapeDtypeStruct((NP, H), jnp.float32),
                   jax.ShapeDtypeStruct((NP, 1), jnp.float32),
                   jax.ShapeDtypeStruct((NP, 1), jnp.float32)],
    )(xp, w1, b1.reshape(1, H), dega, degb)


def _tc_layer(psum, h, x0, dis_c, dis2_c, w, beta):
    def body(p_ref, h_ref, x0_ref, dis_ref, dis2_ref, w_ref,
             hn_ref, hs_ref):
        agg = dis_ref[...] * p_ref[...] + dis2_ref[...] * h_ref[...]
        hh = (1.0 - ALPHA) * agg + ALPHA * x0_ref[...]
        m = jnp.dot(hh, w_ref[...], preferred_element_type=jnp.float32)
        hn = jnp.maximum((1.0 - beta) * hh + beta * m, 0.0)
        hn_ref[...] = hn
        hs_ref[...] = hn * dis_ref[...]

    return pl.pallas_call(
        body,
        grid=(NP // RB,),
        in_specs=[
            pl.BlockSpec((RB, H), lambda i: (i, 0)),
            pl.BlockSpec((RB, H), lambda i: (i, 0)),
            pl.BlockSpec((RB, H), lambda i: (i, 0)),
            pl.BlockSpec((RB, 1), lambda i: (i, 0)),
            pl.BlockSpec((RB, 1), lambda i: (i, 0)),
            pl.BlockSpec((H, H), lambda i: (0, 0)),
        ],
        out_specs=[pl.BlockSpec((RB, H), lambda i: (i, 0)),
                   pl.BlockSpec((RB, H), lambda i: (i, 0))],
        out_shape=[jax.ShapeDtypeStruct((NP, H), jnp.float32),
                   jax.ShapeDtypeStruct((NP, H), jnp.float32)],
    )(psum, h, x0, dis_c, dis2_c, w)


def _tc_lin2(h, w2, b2):
    def body(h_ref, w_ref, b_ref, o_ref):
        o_ref[...] = (jnp.dot(h_ref[...], w_ref[...],
                              preferred_element_type=jnp.float32)
                      + b_ref[...])

    return pl.pallas_call(
        body,
        grid=(NP // RB,),
        in_specs=[
            pl.BlockSpec((RB, H), lambda i: (i, 0)),
            pl.BlockSpec((H, DOUT), lambda i: (0, 0)),
            pl.BlockSpec((1, DOUT), lambda i: (0, 0)),
        ],
        out_specs=pl.BlockSpec((RB, DOUT), lambda i: (i, 0)),
        out_shape=jax.ShapeDtypeStruct((NP, DOUT), jnp.float32),
    )(h, w2, b2.reshape(1, DOUT))


def kernel(x, edge_index, w_lin1, b_lin1, conv_w, w_lin2, b_lin2):
    xp = jnp.pad(x, ((0, NP - N), (0, 0)))
    pad = EP - E
    rowp = jnp.concatenate(
        [edge_index[0], jnp.full((pad,), NP - 1, jnp.int32)])
    colp = jnp.concatenate(
        [edge_index[1], jnp.full((pad,), NP - 1, jnp.int32)])
    row_w = rowp.reshape(NS, NCH2, K)
    col_w = colp.reshape(NS, NCH2, K)

    degs = _sc_degree(colp.reshape(NW, NCH, K))
    dega = degs[:NP].reshape(NP, 1)
    degb = degs[NP:].reshape(NP, 1)

    h, hs, dis_c, dis2_c = _tc_lin1(xp, w_lin1, b_lin1, dega, degb)
    x0 = h
    for i in range(L):
        beta = math.log(THETA / (i + 1) + 1.0)
        psum = _sc_scatter_partials(hs, row_w, col_w)
        h, hs = _tc_layer(psum, h, x0, dis_c, dis2_c, conv_w[i], beta)

    out = _tc_lin2(h, w_lin2, b_lin2)
    return out[:N]
